# baseline (speedup 1.0000x reference)
import functools

import numpy as np

import jax
import jax.numpy as jnp
from jax import lax
from jax.experimental import pallas as pl
from jax.experimental.pallas import tpu as pltpu
from jax.experimental.pallas import tpu_sc as plsc

N = 10000
E = 320000
D = 128
EDIM = 16

NC = 2
NS = 16
NW = NC * NS
L = 16

CH = 128
KMAX = 80
KH = KMAX // 2
NCHUNK_P = KMAX * NW
E_PAD = NCHUNK_P * CH
PADE = E_PAD - E

NPT = 624
ZROWS = 208
NZ = NPT // ZROWS
TBASE = NS * NPT
OTAIL = N - TBASE

SPAN = E_PAD // NW

_mesh = lambda: plsc.VectorSubcoreMesh(
    core_axis_name="c", subcore_axis_name="s", num_cores=NC, num_subcores=NS)


def _widx():
    c = lax.axis_index("c")
    s = lax.axis_index("s")
    return c, s, s * NC + c


_PERM = (np.arange(KMAX)[None, :] * NW + np.arange(NW)[:, None]).reshape(-1)


@functools.partial(
    pl.kernel,
    out_type=jax.ShapeDtypeStruct((NC * N, D), jnp.float32),
    mesh=_mesh(),
    scratch_types=[
        pltpu.VMEM((2, 2, CH), jnp.int32),
        pltpu.VMEM((2, CH, D), jnp.float32),
        pltpu.VMEM_SHARED((N, D), jnp.float32),
        pltpu.SemaphoreType.DMA,
        pltpu.SemaphoreType.DMA,
        pltpu.SemaphoreType.DMA,
        pltpu.SemaphoreType.DMA,
    ],
)
def _sc_segsum_rows(table, srcp, dstp, zh,
                    sh_out, idx, rows, acc_s, sg0, sg1, si0, si1):
    c, s, wid = _widx()
    for z in range(NZ):
        pltpu.sync_copy(zh, acc_s.at[pl.ds(s * NPT + z * ZROWS, ZROWS)])

    @pl.when(s == NS - 1)
    def _():
        pltpu.sync_copy(zh.at[pl.ds(0, OTAIL)], acc_s.at[pl.ds(TBASE, OTAIL)])

    plsc.subcore_barrier()

    def step(k, carry):
        row = k * NW + wid
        pltpu.sync_copy(srcp.at[row], idx.at[0, 0])
        pltpu.sync_copy(dstp.at[row], idx.at[0, 1])
        pltpu.async_copy(table.at[idx.at[0, 0]], rows.at[0], sg0).wait()
        pltpu.sync_copy(rows.at[0], acc_s.at[idx.at[0, 1]], add=True)
        return carry

    lax.fori_loop(0, KMAX, step, 0)
    plsc.subcore_barrier()
    base = c * N + s * NPT
    pltpu.sync_copy(acc_s.at[pl.ds(s * NPT, NPT)], sh_out.at[pl.ds(base, NPT)])

    @pl.when(s == NS - 1)
    def _():
        tb = c * N + TBASE
        pltpu.sync_copy(acc_s.at[pl.ds(TBASE, OTAIL)], sh_out.at[pl.ds(tb, OTAIL)])


@functools.partial(
    pl.kernel,
    out_type=jax.ShapeDtypeStruct((NC * N, D), jnp.float32),
    mesh=_mesh(),
    scratch_types=[
        pltpu.VMEM((KMAX, CH), jnp.int32),
        pltpu.VMEM((CH, EDIM), jnp.float32),
        pltpu.VMEM((CH, D), jnp.float32),
        pltpu.VMEM_SHARED((N, D), jnp.float32),
    ],
)
def _sc_ef_cnt(dstp, ef, tmpl, zh,
               ec_out, dst_i, efv, buf, acc):
    c, s, wid = _widx()
    for z in range(NZ):
        pltpu.sync_copy(zh, acc.at[pl.ds(s * NPT + z * ZROWS, ZROWS)])

    @pl.when(s == NS - 1)
    def _():
        pltpu.sync_copy(zh.at[pl.ds(0, OTAIL)], acc.at[pl.ds(TBASE, OTAIL)])

    pltpu.sync_copy(tmpl, buf)
    plsc.subcore_barrier()

    pltpu.sync_copy(dstp.at[pl.ds(wid * KMAX, KMAX)], dst_i)

    def step(k, carry):
        row = k * NW + wid
        pltpu.sync_copy(ef.at[pl.ds(row * CH, CH)], efv)
        for j in range(CH):
            buf[j, pl.ds(0, EDIM)] = efv[j]
        pltpu.sync_copy(buf, acc.at[dst_i.at[k]], add=True)
        return carry

    lax.fori_loop(0, KMAX, step, 0)
    plsc.subcore_barrier()
    base = c * N + s * NPT
    pltpu.sync_copy(acc.at[pl.ds(s * NPT, NPT)], ec_out.at[pl.ds(base, NPT)])

    @pl.when(s == NS - 1)
    def _():
        tb = c * N + TBASE
        pltpu.sync_copy(acc.at[pl.ds(TBASE, OTAIL)], ec_out.at[pl.ds(tb, OTAIL)])


@functools.partial(
    pl.kernel,
    out_type=(
        jax.ShapeDtypeStruct((E_PAD,), jnp.float32),
        jax.ShapeDtypeStruct((E_PAD,), jnp.float32),
    ),
    mesh=_mesh(),
    compiler_params=pltpu.CompilerParams(needs_layout_passes=False),
    scratch_types=[
        pltpu.VMEM((N + L,), jnp.float32),
        pltpu.VMEM((N + L,), jnp.float32),
        pltpu.VMEM((N + L,), jnp.float32),
        pltpu.VMEM((N + L,), jnp.float32),
        pltpu.VMEM((SPAN,), jnp.int32),
        pltpu.VMEM((SPAN,), jnp.int32),
        pltpu.VMEM((SPAN,), jnp.float32),
        pltpu.VMEM((SPAN,), jnp.float32),
        pltpu.VMEM((SPAN,), jnp.float32),
        pltpu.VMEM((SPAN,), jnp.float32),
    ],
)
def _sc_edge_score(su0, su1, sv0, sv1, srcf, dstf, es0, es1,
                   o0, o1,
                   su0_v, su1_v, sv0_v, sv1_v, src_v, dst_v,
                   e0v, e1v, o0v, o1v):
    c, s, wid = _widx()
    pltpu.sync_copy(su0, su0_v.at[pl.ds(0, N)])
    pltpu.sync_copy(su1, su1_v.at[pl.ds(0, N)])
    pltpu.sync_copy(sv0, sv0_v.at[pl.ds(0, N)])
    pltpu.sync_copy(sv1, sv1_v.at[pl.ds(0, N)])
    base = wid * SPAN
    pltpu.sync_copy(srcf.at[pl.ds(base, SPAN)], src_v)
    pltpu.sync_copy(dstf.at[pl.ds(base, SPAN)], dst_v)
    pltpu.sync_copy(es0.at[pl.ds(base, SPAN)], e0v)
    pltpu.sync_copy(es1.at[pl.ds(base, SPAN)], e1v)

    def step(i, carry):
        sl = pl.ds(i * L, L)
        si = src_v[sl]
        di = dst_v[sl]
        o0v[sl] = (plsc.load_gather(su0_v, [si])
                   + plsc.load_gather(sv0_v, [di]) + e0v[sl])
        o1v[sl] = (plsc.load_gather(su1_v, [si])
                   + plsc.load_gather(sv1_v, [di]) + e1v[sl])
        return carry

    lax.fori_loop(0, SPAN // L, step, 0)
    pltpu.sync_copy(o0v, o0.at[pl.ds(base, SPAN)])
    pltpu.sync_copy(o1v, o1.at[pl.ds(base, SPAN)])


_BN = 400
_GN = N // _BN


def _layer_body(x, p0, p1, e0, e1, csh, cec, wmh, wme, bm, wat, wab, ba, wuv,
                h_out, suv_out):
    f32 = jnp.float32
    rid = lax.broadcasted_iota(jnp.int32, (_BN, 1), 0)
    first = jnp.where((rid == 0) & (pl.program_id(0) == 0), 1.0, 0.0)
    sh = p0[...] + p1[...] - first * csh[...]
    ec = e0[...] + e1[...] - first * cec[...]
    se = ec[:, :EDIM]
    cnt = ec[:, EDIM:EDIM + 1]
    agg = (jnp.dot(sh, wmh[...], preferred_element_type=f32)
           + jnp.dot(se, wme[...], preferred_element_type=f32)
           + cnt * bm[...])
    h_neigh = agg / jnp.maximum(cnt, 1.0)
    h = jnp.maximum(
        jnp.dot(x[...], wat[...], preferred_element_type=f32)
        + jnp.dot(h_neigh, wab[...], preferred_element_type=f32)
        + ba[...], 0.0)
    h_out[...] = h
    suv_out[...] = jnp.dot(h, wuv[...], preferred_element_type=f32)


def _layer_tc(x, p0, p1, e0, e1, csh, cec, wmh, wme, bm, wat, wab, ba, wuv):
    row = lambda i: (i, 0)
    fix = lambda i: (0, 0)
    return pl.pallas_call(
        _layer_body,
        grid=(_GN,),
        in_specs=[
            pl.BlockSpec((_BN, D), row),
            pl.BlockSpec((_BN, D), row),
            pl.BlockSpec((_BN, D), row),
            pl.BlockSpec((_BN, D), row),
            pl.BlockSpec((_BN, D), row),
            pl.BlockSpec((1, D), fix),
            pl.BlockSpec((1, D), fix),
            pl.BlockSpec((D, D), fix),
            pl.BlockSpec((EDIM, D), fix),
            pl.BlockSpec((1, D), fix),
            pl.BlockSpec((D, D), fix),
            pl.BlockSpec((D, D), fix),
            pl.BlockSpec((1, D), fix),
            pl.BlockSpec((D, 8), fix),
        ],
        out_specs=[
            pl.BlockSpec((_BN, D), row),
            pl.BlockSpec((_BN, 8), row),
        ],
        out_shape=[
            jax.ShapeDtypeStruct((N, D), jnp.float32),
            jax.ShapeDtypeStruct((N, 8), jnp.float32),
        ],
    )(x, p0, p1, e0, e1, csh, cec, wmh, wme, bm, wat, wab, ba, wuv)


_BE = 3200
_GE = E // _BE


def _edge_dense_body(ef, wemb, bemb, wpe, bp, es_out):
    f32 = jnp.float32
    ee = jnp.maximum(
        jnp.dot(ef[...], wemb[...], preferred_element_type=f32) + bemb[...], 0.0)
    es_out[...] = jnp.dot(ee, wpe[...], preferred_element_type=f32) + bp[...]


def _edge_dense_tc(ef, wemb, bemb, wpe, bp):
    row = lambda i: (i, 0)
    fix = lambda i: (0, 0)
    return pl.pallas_call(
        _edge_dense_body,
        grid=(_GE,),
        in_specs=[
            pl.BlockSpec((_BE, EDIM), row),
            pl.BlockSpec((EDIM, 32), fix),
            pl.BlockSpec((1, 32), fix),
            pl.BlockSpec((32, 8), fix),
            pl.BlockSpec((1, 8), fix),
        ],
        out_specs=pl.BlockSpec((_BE, 8), row),
        out_shape=jax.ShapeDtypeStruct((E, 8), jnp.float32),
    )(ef, wemb, bemb, wpe, bp)


def kernel(nfeats, efeats, W_msg1, b_msg1, W_app1, b_app1, W_msg2, b_msg2,
           W_app2, b_app2, W_emb, b_emb, W_pred, b_pred, edge_index):
    f32 = jnp.float32
    i32 = jnp.int32
    src_f = jnp.concatenate([edge_index[0], jnp.zeros((PADE,), i32)])
    dst_f = jnp.concatenate([edge_index[1], jnp.zeros((PADE,), i32)])
    srcp = src_f.reshape(NCHUNK_P, CH)
    dstp = dst_f.reshape(NCHUNK_P, CH)
    dstp_w = dstp[_PERM]
    ef_pad = jnp.concatenate([efeats, jnp.zeros((PADE, EDIM), f32)])
    zh = jnp.zeros((ZROWS, D), f32)
    tmpl = jnp.zeros((CH, D), f32).at[:, EDIM:2 * EDIM].set(1.0)

    wuv = jnp.concatenate(
        [W_pred[:D], W_pred[D:2 * D], jnp.zeros((D, 4), f32)], axis=1)
    wpe = jnp.concatenate([W_pred[2 * D:], jnp.zeros((32, 6), f32)], axis=1)
    bp = jnp.concatenate([b_pred, jnp.zeros((6,), f32)]).reshape(1, 8)

    cec = jnp.zeros((1, D), f32).at[0, EDIM].set(float(PADE))

    sh_p = _sc_segsum_rows(nfeats, srcp, dstp, zh)
    ec_p = _sc_ef_cnt(dstp_w, ef_pad, tmpl, zh)

    h1, _ = _layer_tc(
        nfeats, sh_p[:N], sh_p[N:], ec_p[:N], ec_p[N:],
        float(PADE) * nfeats[0:1], cec,
        W_msg1[:D], W_msg1[D:], b_msg1.reshape(1, D),
        W_app1[:D], W_app1[D:], b_app1.reshape(1, D), wuv)

    sh2_p = _sc_segsum_rows(h1, srcp, dstp, zh)
    _, suv = _layer_tc(
        h1, sh2_p[:N], sh2_p[N:], ec_p[:N], ec_p[N:],
        float(PADE) * h1[0:1], cec,
        W_msg2[:D], W_msg2[D:], b_msg2.reshape(1, D),
        W_app2[:D], W_app2[D:], b_app2.reshape(1, D), wuv)

    es = _edge_dense_tc(efeats, W_emb, b_emb.reshape(1, 32), wpe, bp)
    es0 = jnp.concatenate([es[:, 0], jnp.zeros((PADE,), f32)])
    es1 = jnp.concatenate([es[:, 1], jnp.zeros((PADE,), f32)])

    o0, o1 = _sc_edge_score(
        suv[:, 0], suv[:, 1], suv[:, 2], suv[:, 3],
        src_f, dst_f, es0, es1)
    return jnp.stack([o0[:E], o1[:E]], axis=1)


# speedup vs baseline: 1.5584x; 1.5584x over previous
import functools

import numpy as np

import jax
import jax.numpy as jnp
from jax import lax
from jax.experimental import pallas as pl
from jax.experimental.pallas import tpu as pltpu
from jax.experimental.pallas import tpu_sc as plsc

N = 10000
E = 320000
D = 128
EDIM = 16

NC = 2
NS = 16
NW = NC * NS
L = 16

CH = 128
NCHUNK = E // CH
KMAX = 80
NCHUNK_P = KMAX * NW
E_PAD = NCHUNK_P * CH
PADE = E_PAD - E

NPT = 624
ZROWS = 208
NZ = NPT // ZROWS
TBASE = NS * NPT
OTAIL = N - TBASE

SPAN = E_PAD // NW

_mesh = lambda: plsc.VectorSubcoreMesh(
    core_axis_name="c", subcore_axis_name="s", num_cores=NC, num_subcores=NS)


def _widx():
    c = lax.axis_index("c")
    s = lax.axis_index("s")
    return c, s, s * NC + c


_PERM = (np.arange(KMAX)[None, :] * NW + np.arange(NW)[:, None]).reshape(-1)


@functools.partial(
    pl.kernel,
    out_type=jax.ShapeDtypeStruct((NC * N, D), jnp.float32),
    mesh=_mesh(),
    scratch_types=[
        pltpu.VMEM((2, 2, CH), jnp.int32),
        pltpu.VMEM((2, CH, D), jnp.float32),
        pltpu.VMEM_SHARED((N, D), jnp.float32),
        pltpu.SemaphoreType.DMA,
        pltpu.SemaphoreType.DMA,
        pltpu.SemaphoreType.DMA,
        pltpu.SemaphoreType.DMA,
    ],
)
def _sc_segsum_rows(table, srcp, dstp, zh,
                    sh_out, idx, rows, acc_s, sg0, sg1, si0, si1):
    c, s, wid = _widx()
    for z in range(NZ):
        pltpu.sync_copy(zh, acc_s.at[pl.ds(s * NPT + z * ZROWS, ZROWS)])

    @pl.when(s == NS - 1)
    def _():
        pltpu.sync_copy(zh.at[pl.ds(0, OTAIL)], acc_s.at[pl.ds(TBASE, OTAIL)])

    plsc.subcore_barrier()

    def step(k, carry):
        row = k * NW + wid

        @pl.when(row < NCHUNK)
        def _():
            pltpu.sync_copy(srcp.at[row], idx.at[0, 0])
            pltpu.sync_copy(dstp.at[row], idx.at[0, 1])
            pltpu.async_copy(table.at[idx.at[0, 0]], rows.at[0], sg0).wait()
            pltpu.sync_copy(rows.at[0], acc_s.at[idx.at[0, 1]], add=True)

        return carry

    lax.fori_loop(0, KMAX, step, 0)
    plsc.subcore_barrier()
    base = c * N + s * NPT
    pltpu.sync_copy(acc_s.at[pl.ds(s * NPT, NPT)], sh_out.at[pl.ds(base, NPT)])

    @pl.when(s == NS - 1)
    def _():
        tb = c * N + TBASE
        pltpu.sync_copy(acc_s.at[pl.ds(TBASE, OTAIL)], sh_out.at[pl.ds(tb, OTAIL)])


@functools.partial(
    pl.kernel,
    out_type=jax.ShapeDtypeStruct((NC * N, D), jnp.float32),
    mesh=_mesh(),
    scratch_types=[
        pltpu.VMEM((KMAX, CH), jnp.int32),
        pltpu.VMEM((CH, EDIM), jnp.float32),
        pltpu.VMEM((CH, D), jnp.float32),
        pltpu.VMEM_SHARED((N, D), jnp.float32),
    ],
)
def _sc_ef_cnt(dstp, ef, tmpl, zh,
               ec_out, dst_i, efv, buf, acc):
    c, s, wid = _widx()
    for z in range(NZ):
        pltpu.sync_copy(zh, acc.at[pl.ds(s * NPT + z * ZROWS, ZROWS)])

    @pl.when(s == NS - 1)
    def _():
        pltpu.sync_copy(zh.at[pl.ds(0, OTAIL)], acc.at[pl.ds(TBASE, OTAIL)])

    pltpu.sync_copy(tmpl, buf)
    plsc.subcore_barrier()

    pltpu.sync_copy(dstp.at[pl.ds(wid * KMAX, KMAX)], dst_i)

    def step(k, carry):
        row = k * NW + wid

        @pl.when(row < NCHUNK)
        def _():
            pltpu.sync_copy(ef.at[pl.ds(row * CH, CH)], efv)
            for j in range(CH):
                buf[j, pl.ds(0, EDIM)] = efv[j]
            pltpu.sync_copy(buf, acc.at[dst_i.at[k]], add=True)

        return carry

    lax.fori_loop(0, KMAX, step, 0)
    plsc.subcore_barrier()
    base = c * N + s * NPT
    pltpu.sync_copy(acc.at[pl.ds(s * NPT, NPT)], ec_out.at[pl.ds(base, NPT)])

    @pl.when(s == NS - 1)
    def _():
        tb = c * N + TBASE
        pltpu.sync_copy(acc.at[pl.ds(TBASE, OTAIL)], ec_out.at[pl.ds(tb, OTAIL)])


@functools.partial(
    pl.kernel,
    out_type=(
        jax.ShapeDtypeStruct((E_PAD,), jnp.float32),
        jax.ShapeDtypeStruct((E_PAD,), jnp.float32),
    ),
    mesh=_mesh(),
    compiler_params=pltpu.CompilerParams(needs_layout_passes=False),
    scratch_types=[
        pltpu.VMEM((N + L,), jnp.float32),
        pltpu.VMEM((N + L,), jnp.float32),
        pltpu.VMEM((N + L,), jnp.float32),
        pltpu.VMEM((N + L,), jnp.float32),
        pltpu.VMEM((SPAN,), jnp.int32),
        pltpu.VMEM((SPAN,), jnp.int32),
        pltpu.VMEM((SPAN,), jnp.float32),
        pltpu.VMEM((SPAN,), jnp.float32),
        pltpu.VMEM((SPAN,), jnp.float32),
        pltpu.VMEM((SPAN,), jnp.float32),
    ],
)
def _sc_edge_score(su0, su1, sv0, sv1, srcf, dstf, es0, es1,
                   o0, o1,
                   su0_v, su1_v, sv0_v, sv1_v, src_v, dst_v,
                   e0v, e1v, o0v, o1v):
    c, s, wid = _widx()
    pltpu.sync_copy(su0, su0_v.at[pl.ds(0, N)])
    pltpu.sync_copy(su1, su1_v.at[pl.ds(0, N)])
    pltpu.sync_copy(sv0, sv0_v.at[pl.ds(0, N)])
    pltpu.sync_copy(sv1, sv1_v.at[pl.ds(0, N)])
    base = wid * SPAN
    pltpu.sync_copy(srcf.at[pl.ds(base, SPAN)], src_v)
    pltpu.sync_copy(dstf.at[pl.ds(base, SPAN)], dst_v)
    pltpu.sync_copy(es0.at[pl.ds(base, SPAN)], e0v)
    pltpu.sync_copy(es1.at[pl.ds(base, SPAN)], e1v)

    def step(i, carry):
        sl = pl.ds(i * L, L)
        si = src_v[sl]
        di = dst_v[sl]
        o0v[sl] = (plsc.load_gather(su0_v, [si])
                   + plsc.load_gather(sv0_v, [di]) + e0v[sl])
        o1v[sl] = (plsc.load_gather(su1_v, [si])
                   + plsc.load_gather(sv1_v, [di]) + e1v[sl])
        return carry

    lax.fori_loop(0, SPAN // L, step, 0)
    pltpu.sync_copy(o0v, o0.at[pl.ds(base, SPAN)])
    pltpu.sync_copy(o1v, o1.at[pl.ds(base, SPAN)])


_BN = 400
_GN = N // _BN


def _layer_body(x, p0, p1, e0, e1, wmh, wme, bm, wat, wab, ba, wuv,
                h_out, suv_out):
    f32 = jnp.float32
    sh = p0[...] + p1[...]
    ec = e0[...] + e1[...]
    se = ec[:, :EDIM]
    cnt = ec[:, EDIM:EDIM + 1]
    agg = (jnp.dot(sh, wmh[...], preferred_element_type=f32)
           + jnp.dot(se, wme[...], preferred_element_type=f32)
           + cnt * bm[...])
    h_neigh = agg / jnp.maximum(cnt, 1.0)
    h = jnp.maximum(
        jnp.dot(x[...], wat[...], preferred_element_type=f32)
        + jnp.dot(h_neigh, wab[...], preferred_element_type=f32)
        + ba[...], 0.0)
    h_out[...] = h
    suv_out[...] = jnp.dot(h, wuv[...], preferred_element_type=f32)


def _layer_tc(x, p0, p1, e0, e1, wmh, wme, bm, wat, wab, ba, wuv):
    row = lambda i: (i, 0)
    fix = lambda i: (0, 0)
    return pl.pallas_call(
        _layer_body,
        grid=(_GN,),
        in_specs=[
            pl.BlockSpec((_BN, D), row),
            pl.BlockSpec((_BN, D), row),
            pl.BlockSpec((_BN, D), row),
            pl.BlockSpec((_BN, D), row),
            pl.BlockSpec((_BN, D), row),
            pl.BlockSpec((D, D), fix),
            pl.BlockSpec((EDIM, D), fix),
            pl.BlockSpec((1, D), fix),
            pl.BlockSpec((D, D), fix),
            pl.BlockSpec((D, D), fix),
            pl.BlockSpec((1, D), fix),
            pl.BlockSpec((D, 8), fix),
        ],
        out_specs=[
            pl.BlockSpec((_BN, D), row),
            pl.BlockSpec((_BN, 8), row),
        ],
        out_shape=[
            jax.ShapeDtypeStruct((N, D), jnp.float32),
            jax.ShapeDtypeStruct((N, 8), jnp.float32),
        ],
    )(x, p0, p1, e0, e1, wmh, wme, bm, wat, wab, ba, wuv)


_BE = 3200
_GE = E // _BE


def _edge_dense_body(ef, wemb, bemb, wpe, bp, es_out):
    f32 = jnp.float32
    ee = jnp.maximum(
        jnp.dot(ef[...], wemb[...], preferred_element_type=f32) + bemb[...], 0.0)
    es_out[...] = jnp.dot(ee, wpe[...], preferred_element_type=f32) + bp[...]


def _edge_dense_tc(ef, wemb, bemb, wpe, bp):
    row = lambda i: (i, 0)
    fix = lambda i: (0, 0)
    return pl.pallas_call(
        _edge_dense_body,
        grid=(_GE,),
        in_specs=[
            pl.BlockSpec((_BE, EDIM), row),
            pl.BlockSpec((EDIM, 32), fix),
            pl.BlockSpec((1, 32), fix),
            pl.BlockSpec((32, 8), fix),
            pl.BlockSpec((1, 8), fix),
        ],
        out_specs=pl.BlockSpec((_BE, 8), row),
        out_shape=jax.ShapeDtypeStruct((E, 8), jnp.float32),
    )(ef, wemb, bemb, wpe, bp)


def kernel(nfeats, efeats, W_msg1, b_msg1, W_app1, b_app1, W_msg2, b_msg2,
           W_app2, b_app2, W_emb, b_emb, W_pred, b_pred, edge_index):
    f32 = jnp.float32
    i32 = jnp.int32
    src_f = jnp.concatenate([edge_index[0], jnp.zeros((PADE,), i32)])
    dst_f = jnp.concatenate([edge_index[1], jnp.zeros((PADE,), i32)])
    srcp = src_f.reshape(NCHUNK_P, CH)
    dstp = dst_f.reshape(NCHUNK_P, CH)
    dstp_w = dstp[_PERM]
    ef_pad = jnp.concatenate([efeats, jnp.zeros((PADE, EDIM), f32)])
    zh = jnp.zeros((ZROWS, D), f32)
    tmpl = jnp.zeros((CH, D), f32).at[:, EDIM:2 * EDIM].set(1.0)

    wuv = jnp.concatenate(
        [W_pred[:D], W_pred[D:2 * D], jnp.zeros((D, 4), f32)], axis=1)
    wpe = jnp.concatenate([W_pred[2 * D:], jnp.zeros((32, 6), f32)], axis=1)
    bp = jnp.concatenate([b_pred, jnp.zeros((6,), f32)]).reshape(1, 8)

    sh_p = _sc_segsum_rows(nfeats, srcp, dstp, zh)
    ec_p = _sc_ef_cnt(dstp_w, ef_pad, tmpl, zh)

    h1, _ = _layer_tc(
        nfeats, sh_p[:N], sh_p[N:], ec_p[:N], ec_p[N:],
        W_msg1[:D], W_msg1[D:], b_msg1.reshape(1, D),
        W_app1[:D], W_app1[D:], b_app1.reshape(1, D), wuv)

    sh2_p = _sc_segsum_rows(h1, srcp, dstp, zh)
    _, suv = _layer_tc(
        h1, sh2_p[:N], sh2_p[N:], ec_p[:N], ec_p[N:],
        W_msg2[:D], W_msg2[D:], b_msg2.reshape(1, D),
        W_app2[:D], W_app2[D:], b_app2.reshape(1, D), wuv)

    es = _edge_dense_tc(efeats, W_emb, b_emb.reshape(1, 32), wpe, bp)
    es0 = jnp.concatenate([es[:, 0], jnp.zeros((PADE,), f32)])
    es1 = jnp.concatenate([es[:, 1], jnp.zeros((PADE,), f32)])

    o0, o1 = _sc_edge_score(
        suv[:, 0], suv[:, 1], suv[:, 2], suv[:, 3],
        src_f, dst_f, es0, es1)
    return jnp.stack([o0[:E], o1[:E]], axis=1)
